# baseline (speedup 1.0000x reference)
import jax
import jax.numpy as jnp
from jax import lax
from jax.experimental import pallas as pl
from jax.experimental.pallas import tpu as pltpu
from jax.experimental.pallas import tpu_sc as plsc

TASKS = 1_000_000
DIM = 16
BATCH = 16384
CLIP = 1.0

_info = plsc.get_sparse_core_info()
_NC, _NS, _L = _info.num_cores, _info.num_subcores, _info.num_lanes
_NW = _NC * _NS
_BPW = BATCH // _NW
_NSEM = 8


def _sc_body(tbl_hbm, idx_hbm, out_hbm, idx_v, rows_v, *sems):
    wid = lax.axis_index("s") * _NC + lax.axis_index("c")
    base = wid * _BPW
    pltpu.sync_copy(idx_hbm.at[pl.ds(base, _BPW)], idx_v)

    def issue(c, _):
        o = pl.multiple_of(c * 16, 16)
        v = idx_v[pl.ds(o, 16)]
        for j in range(16):
            pltpu.async_copy(
                tbl_hbm.at[v[j]], rows_v.at[o + j], sems[j % _NSEM]
            )
        return 0

    lax.fori_loop(0, _BPW // 16, issue, 0)

    def drain(c, _):
        for s in range(_NSEM):
            pltpu.make_async_copy(tbl_hbm.at[0], rows_v.at[0], sems[s]).wait()
        return 0

    lax.fori_loop(0, _BPW // _NSEM, drain, 0)

    def clip_rows(i, _):
        o = pl.multiple_of(i * 8, 8)
        for j in range(8):
            rows_v[o + j] = jnp.minimum(jnp.maximum(rows_v[o + j], -CLIP), CLIP)
        return 0

    lax.fori_loop(0, _BPW // 8, clip_rows, 0)

    pltpu.sync_copy(rows_v, out_hbm.at[pl.ds(base, _BPW)])


@jax.jit
def _gather_clip(table, idx_flat):
    mesh = plsc.VectorSubcoreMesh(core_axis_name="c", subcore_axis_name="s")
    kfn = pl.kernel(
        _sc_body,
        mesh=mesh,
        out_type=jax.ShapeDtypeStruct((BATCH, DIM), jnp.float32),
        scratch_types=[
            pltpu.VMEM((_BPW,), jnp.int32),
            pltpu.VMEM((_BPW, DIM), jnp.float32),
        ]
        + [pltpu.SemaphoreType.DMA] * _NSEM,
        compiler_params=pltpu.CompilerParams(use_tc_tiling_on_sc=True),
    )
    return kfn(table, idx_flat)


def kernel(idx, context_hat):
    return _gather_clip(context_hat, idx[..., 0])
